# Initial kernel scaffold; baseline (speedup 1.0000x reference)
import jax
import jax.numpy as jnp
from jax.experimental import pallas as pl


def kernel(feats, batch):
    raise NotImplementedError("write your pallas kernel here")


# speedup vs baseline: 3.4859x; 3.4859x over previous
import jax
import jax.numpy as jnp
from jax import lax
from jax.experimental import pallas as pl
from jax.experimental.pallas import tpu as pltpu
from jax.experimental.pallas import tpu_sc as plsc

NUM_SEGS = 10000
SEGS_PAD = 10240
ROWS = 160000
D = 256
HALF = 128
CHUNK = 80
N_SUB = 16
LANES = 16
ROWS_PER_SUB = ROWS // N_SUB
CHUNKS = ROWS_PER_SUB // CHUNK
SEGS_PER_SUB = SEGS_PAD // N_SUB
SB = 128
SBLOCKS = SEGS_PER_SUB // SB
CROWS = SEGS_PAD // HALF


def _pool_mean_sc(feats_hbm, batch_hbm, out_hbm, acc_sh, cnt_sh, idx_v,
                  rows_v, cnt_local, idrow_v, seg_v, cnt_v, inv_v):
    cid = lax.axis_index("c")
    sid = lax.axis_index("s")
    col0 = cid * HALF

    def zero_seg_row(i, carry):
        for v in range(HALF // LANES):
            seg_v[i, pl.ds(v * LANES, LANES)] = jnp.zeros((LANES,), jnp.float32)
        return carry

    lax.fori_loop(0, SB, zero_seg_row, 0)

    def zero_cnt_row(i, carry):
        for v in range(HALF // LANES):
            cnt_local[i, pl.ds(v * LANES, LANES)] = jnp.zeros(
                (LANES,), jnp.float32)
        return carry

    lax.fori_loop(0, CROWS, zero_cnt_row, 0)

    for k in range(CROWS // LANES):
        idrow_v[pl.ds(k * LANES, LANES)] = (
            lax.iota(jnp.int32, LANES) + k * LANES)

    for b in range(SBLOCKS):
        pltpu.sync_copy(seg_v, acc_sh.at[pl.ds(sid * SEGS_PER_SUB + b * SB, SB)])

    @pl.when(sid == 0)
    def _():
        pltpu.sync_copy(cnt_local, cnt_sh)

    plsc.subcore_barrier()

    row_base = sid * ROWS_PER_SUB
    ones16 = jnp.ones((LANES,), jnp.float32)

    def chunk_body(j, carry):
        base = row_base + j * CHUNK
        pltpu.sync_copy(batch_hbm.at[pl.ds(base, CHUNK)], idx_v)
        pltpu.sync_copy(feats_hbm.at[pl.ds(base, CHUNK), pl.ds(col0, HALF)],
                        rows_v)
        pltpu.sync_copy(rows_v, acc_sh.at[idx_v], add=True)
        for k in range(CHUNK // LANES):
            seg = idx_v[pl.ds(k * LANES, LANES)]
            row = lax.shift_right_logical(seg, jnp.full((LANES,), 7, jnp.int32))
            col = lax.bitwise_and(seg, jnp.full((LANES,), HALF - 1, jnp.int32))
            plsc.addupdate_scatter(cnt_local, [row, col], ones16)
        return carry

    lax.fori_loop(0, CHUNKS, chunk_body, 0)

    pltpu.sync_copy(cnt_local, cnt_sh.at[idrow_v], add=True)
    plsc.subcore_barrier()

    for b in range(SBLOCKS):
        seg0 = sid * SEGS_PER_SUB + b * SB
        crow = sid * SBLOCKS + b
        pltpu.sync_copy(acc_sh.at[pl.ds(seg0, SB)], seg_v)
        pltpu.sync_copy(cnt_sh.at[crow], cnt_v)

        for k in range(HALF // LANES):
            c = cnt_v[pl.ds(k * LANES, LANES)]
            inv_v[pl.ds(k * LANES, LANES)] = 1.0 / jnp.maximum(c, 1.0)

        def mean_row(i, carry):
            iv = plsc.load_gather(inv_v, [jnp.full((LANES,), i, jnp.int32)])
            for v in range(HALF // LANES):
                seg_v[i, pl.ds(v * LANES, LANES)] = (
                    seg_v[i, pl.ds(v * LANES, LANES)] * iv)
            return carry

        lax.fori_loop(0, SB, mean_row, 0)
        pltpu.sync_copy(seg_v, out_hbm.at[pl.ds(seg0, SB), pl.ds(col0, HALF)])


@jax.jit
def kernel(feats, batch):
    batch = batch.astype(jnp.int32)
    mesh = plsc.VectorSubcoreMesh(core_axis_name="c", subcore_axis_name="s")
    fn = pl.kernel(
        _pool_mean_sc,
        mesh=mesh,
        compiler_params=pltpu.CompilerParams(needs_layout_passes=False),
        out_type=jax.ShapeDtypeStruct((SEGS_PAD, D), jnp.float32),
        scratch_types=[
            pltpu.VMEM_SHARED((SEGS_PAD, HALF), jnp.float32),
            pltpu.VMEM_SHARED((CROWS, HALF), jnp.float32),
            pltpu.VMEM((CHUNK,), jnp.int32),
            pltpu.VMEM((CHUNK, HALF), jnp.float32),
            pltpu.VMEM((CROWS, HALF), jnp.float32),
            pltpu.VMEM((CROWS,), jnp.int32),
            pltpu.VMEM((SB, HALF), jnp.float32),
            pltpu.VMEM((HALF,), jnp.float32),
            pltpu.VMEM((HALF,), jnp.float32),
        ],
    )
    return fn(feats, batch)[:NUM_SEGS]
